# Initial kernel scaffold; baseline (speedup 1.0000x reference)
import functools

import jax
import jax.numpy as jnp
from jax import lax
from jax.experimental import pallas as pl
from jax.experimental.pallas import tpu as pltpu
from jax.experimental.pallas import tpu_sc as plsc

_TAU = 0.995
_B, _H, _W = 8, 512, 512
_NC, _NS, _L = 2, 16, 16
_IPC = _B // _NC
_CAP = 8192
_HALF_ROWS = _H // 2
_CHUNK_ROWS = 32
_NCHUNKS = _HALF_ROWS // _CHUNK_ROWS
_VPR = _W // _L


def _decode(idx_ref, i_ref, j_ref, n):
    nv = (n + (_L - 1)) // _L

    def body(v, carry):
        q = idx_ref[pl.ds(v * _L, _L)]
        i_ref[pl.ds(v * _L, _L)] = (q >> 9).astype(jnp.float32)
        j_ref[pl.ds(v * _L, _L)] = (q & (_W - 1)).astype(jnp.float32)
        return carry

    lax.fori_loop(0, nv, body, 0)


def _pair_loop(o_i, o_j, n_out, inner, minmode):
    p0i = o_i[0]
    p0j = o_j[0]
    lanes = lax.iota(jnp.int32, _L)
    nv = (n_out + (_L - 1)) // _L
    init_acc = jnp.float32(1e30) if minmode else jnp.float32(-1.0)

    def outer(ov, best):
        base = ov * _L
        valid = (base + lanes) < n_out
        av_i = jnp.where(valid, o_i[pl.ds(base, _L)], p0i)
        av_j = jnp.where(valid, o_j[pl.ds(base, _L)], p0j)
        acc = jnp.full((_L,), init_acc, jnp.float32)
        for b_i, b_j, n_in in inner:
            def body(k, acc):
                di = av_i - b_i[k]
                dj = av_j - b_j[k]
                d2 = di * di + dj * dj
                return jnp.minimum(acc, d2) if minmode else jnp.maximum(acc, d2)
            acc = lax.fori_loop(0, n_in, body, acc)
        return jnp.maximum(best, acc)

    return lax.fori_loop(0, nv, outer, jnp.full((_L,), -1.0, jnp.float32))


def _hausdorff_sq_kernel(pred_hbm, tgt_hbm, out_hbm,
                         chunk, loc_idx, cnt_buf,
                         a_idx, b_idx0, b_idx1,
                         a_i, a_j, b_i0, b_j0, b_i1, b_j1,
                         counts_v, res_v, res_all, out_v,
                         pts_sh, counts_sh, res_sh):
    c = lax.axis_index("c")
    s = lax.axis_index("s")

    u = s // 2
    h = s % 2
    img = c * _IPC + (u // 2)
    side = u % 2
    row0 = h * _HALF_ROWS

    lanes = lax.iota(jnp.int32, _L)
    offm1 = jnp.full((_L,), -1, jnp.int32)
    for chunk_i in range(_NCHUNKS):
        r_lo = row0 + chunk_i * _CHUNK_ROWS

        @pl.when(side == 0)
        def _():
            pltpu.sync_copy(pred_hbm.at[img, pl.ds(r_lo, _CHUNK_ROWS)], chunk)

        @pl.when(side == 1)
        def _():
            pltpu.sync_copy(tgt_hbm.at[img, pl.ds(r_lo, _CHUNK_ROWS)], chunk)

        def row_body(r, offm1):
            gbase = (r_lo + r) * _W + lanes
            for k in range(_VPR):
                vec = chunk[r, pl.ds(k * _L, _L)]
                m = vec >= _TAU
                pcnt = plsc.all_reduce_population_count(m)
                pos = plsc.cumsum(m.astype(jnp.int32))
                tgt_lane = jnp.minimum(offm1 + pos, _CAP - 1)
                plsc.store_scatter(loc_idx, [tgt_lane], gbase + (k * _L), mask=m)
                offm1 = offm1 + pcnt
            return offm1

        offm1 = lax.fori_loop(0, _CHUNK_ROWS, row_body, offm1)

    cnt_buf[...] = jnp.minimum(offm1 + 1, _CAP)
    pltpu.sync_copy(loc_idx, pts_sh.at[s])
    pltpu.sync_copy(cnt_buf, counts_sh.at[s])
    plsc.subcore_barrier()

    d = s // 2
    h2 = s % 2
    img_loc = d // 2
    dirn = d % 2
    u_a = img_loc * 2 + dirn
    u_b = img_loc * 2 + (1 - dirn)
    row_a = u_a * 2 + h2
    r_b0 = u_b * 2
    r_b1 = u_b * 2 + 1

    pltpu.sync_copy(counts_sh, counts_v)
    pltpu.sync_copy(pts_sh.at[row_a], a_idx)
    pltpu.sync_copy(pts_sh.at[r_b0], b_idx0)
    pltpu.sync_copy(pts_sh.at[r_b1], b_idx1)

    c_ah = counts_v[row_a, 0]
    c_a = counts_v[u_a * 2, 0] + counts_v[u_a * 2 + 1, 0]
    c_b0 = counts_v[r_b0, 0]
    c_b1 = counts_v[r_b1, 0]
    c_b = c_b0 + c_b1

    _decode(a_idx, a_i, a_j, c_ah)
    _decode(b_idx0, b_i0, b_j0, c_b0)
    _decode(b_idx1, b_i1, b_j1, c_b1)

    res_v[...] = jnp.zeros((_L,), jnp.float32)
    inner = ((b_i0, b_j0, c_b0), (b_i1, b_j1, c_b1))

    @pl.when((c_a > 0) & (c_b > 0))
    def _():
        res_v[...] = _pair_loop(a_i, a_j, c_ah, inner, True)

    @pl.when((c_a == 0) & (c_b > 0) & (h2 == 0))
    def _():
        res_v[...] = _pair_loop(b_i0, b_j0, c_b0, inner, False)

    @pl.when((c_a == 0) & (c_b > 0) & (h2 == 1))
    def _():
        res_v[...] = _pair_loop(b_i1, b_j1, c_b1, inner, False)

    pltpu.sync_copy(res_v, res_sh.at[s])
    plsc.subcore_barrier()

    @pl.when(s == 0)
    def _():
        pltpu.sync_copy(res_sh, res_all)
        out_v[...] = jnp.zeros((_L,), jnp.float32)
        for i in range(_IPC):
            v = jnp.maximum(
                jnp.maximum(res_all[4 * i, :], res_all[4 * i + 1, :]),
                jnp.maximum(res_all[4 * i + 2, :], res_all[4 * i + 3, :]),
            )
            out_v[i] = jnp.max(v)
        pltpu.sync_copy(out_v, out_hbm.at[c])


def kernel(prediction, target):
    mesh = plsc.VectorSubcoreMesh(
        core_axis_name="c", subcore_axis_name="s",
        num_cores=_NC, num_subcores=_NS,
    )
    out = pl.kernel(
        _hausdorff_sq_kernel,
        out_type=jax.ShapeDtypeStruct((_NC, _L), jnp.float32),
        mesh=mesh,
        scratch_types=[
            pltpu.VMEM((_CHUNK_ROWS, _W), jnp.float32),
            pltpu.VMEM((_CAP,), jnp.int32),
            pltpu.VMEM((_L,), jnp.int32),
            pltpu.VMEM((_CAP,), jnp.int32),
            pltpu.VMEM((_CAP,), jnp.int32),
            pltpu.VMEM((_CAP,), jnp.int32),
            pltpu.VMEM((_CAP,), jnp.float32),
            pltpu.VMEM((_CAP,), jnp.float32),
            pltpu.VMEM((_CAP,), jnp.float32),
            pltpu.VMEM((_CAP,), jnp.float32),
            pltpu.VMEM((_CAP,), jnp.float32),
            pltpu.VMEM((_CAP,), jnp.float32),
            pltpu.VMEM((_NS, _L), jnp.int32),
            pltpu.VMEM((_L,), jnp.float32),
            pltpu.VMEM((_NS, _L), jnp.float32),
            pltpu.VMEM((_L,), jnp.float32),
            pltpu.VMEM_SHARED((_NS, _CAP), jnp.int32),
            pltpu.VMEM_SHARED((_NS, _L), jnp.int32),
            pltpu.VMEM_SHARED((_NS, _L), jnp.float32),
        ],
    )(prediction, target)
    vals = out[:, :_IPC].reshape(_B)
    return jnp.mean(jnp.sqrt(vals))


# speedup vs baseline: 39.9349x; 39.9349x over previous
import functools

import jax
import jax.numpy as jnp
from jax import lax
from jax.experimental import pallas as pl
from jax.experimental.pallas import tpu as pltpu
from jax.experimental.pallas import tpu_sc as plsc

_TAU = 0.995
_B, _H, _W = 8, 512, 512
_NC, _NS, _L = 2, 16, 16
_IPC = _B // _NC
_CAP = 8192
_HALF_ROWS = _H // 2
_CHUNK_ROWS = 32
_NCHUNKS = _HALF_ROWS // _CHUNK_ROWS
_VPR = _W // _L


def _decode(idx_ref, i_ref, j_ref, n):
    q0 = jnp.min(idx_ref[pl.ds(0, _L)])
    p0i = (q0 >> 9).astype(jnp.float32)
    p0j = (q0 & (_W - 1)).astype(jnp.float32)
    lanes = lax.iota(jnp.int32, _L)
    nv = (n + (_L - 1)) // _L

    def body(v, carry):
        base = v * _L
        q = idx_ref[pl.ds(base, _L)]
        valid = (base + lanes) < n
        i_ref[pl.ds(base, _L)] = jnp.where(valid, (q >> 9).astype(jnp.float32), p0i)
        j_ref[pl.ds(base, _L)] = jnp.where(valid, (q & (_W - 1)).astype(jnp.float32), p0j)
        return carry

    lax.fori_loop(0, nv, body, 0)


def _pair_loop(o_i, o_j, n_out, inner, minmode):
    nv_out = (n_out + (_L - 1)) // _L
    init_acc = jnp.float32(1e30) if minmode else jnp.float32(-1.0)

    def outer(ov, best):
        av_i = o_i[pl.ds(ov * _L, _L)]
        av_j = o_j[pl.ds(ov * _L, _L)]
        acc = jnp.full((_L,), init_acc, jnp.float32)
        for b_i, b_j, n_in in inner:
            nv_in = (n_in + (_L - 1)) // _L

            def ibody(iv, acc):
                bi_vec = b_i[pl.ds(iv * _L, _L)]
                bj_vec = b_j[pl.ds(iv * _L, _L)]
                for lane in range(_L):
                    di = av_i - bi_vec[lane]
                    dj = av_j - bj_vec[lane]
                    d2 = di * di + dj * dj
                    acc = (jnp.minimum(acc, d2) if minmode
                           else jnp.maximum(acc, d2))
                return acc

            acc = lax.fori_loop(0, nv_in, ibody, acc)
        return jnp.maximum(best, acc)

    return lax.fori_loop(0, nv_out, outer, jnp.full((_L,), -1.0, jnp.float32))


def _hausdorff_sq_kernel(src_hbm, out_hbm,
                         chunk, loc_idx, cnt_buf,
                         a_idx, b_idx0, b_idx1,
                         a_i, a_j, b_i0, b_j0, b_i1, b_j1,
                         counts_v, res_iv, res_all, out_v,
                         sh):
    c = lax.axis_index("c")
    s = lax.axis_index("s")

    u = s // 2
    h = s % 2
    img = c * _IPC + (u // 2)
    side = u % 2
    row0 = h * _HALF_ROWS

    lanes = lax.iota(jnp.int32, _L)
    loc_idx[pl.ds(0, _L)] = jnp.full((_L,), 0x7FFFFFFF, jnp.int32)
    offm1 = jnp.full((_L,), -1, jnp.int32)
    for chunk_i in range(_NCHUNKS):
        r_lo = row0 + chunk_i * _CHUNK_ROWS
        pltpu.sync_copy(src_hbm.at[side, img, pl.ds(r_lo, _CHUNK_ROWS)], chunk)

        def row_body(r, offm1):
            gbase = (r_lo + r) * _W + lanes
            for k in range(_VPR):
                vec = chunk[r, pl.ds(k * _L, _L)]
                m = vec >= _TAU
                pcnt = plsc.all_reduce_population_count(m)
                pos = plsc.cumsum(m.astype(jnp.int32))
                tgt_lane = jnp.minimum(offm1 + pos, _CAP - 1)
                plsc.store_scatter(loc_idx, [tgt_lane], gbase + (k * _L), mask=m)
                offm1 = offm1 + pcnt
            return offm1

        offm1 = lax.fori_loop(0, _CHUNK_ROWS, row_body, offm1)

    cnt_buf[...] = jnp.minimum(offm1 + 1, _CAP)
    pltpu.sync_copy(loc_idx, sh.at[s, pl.ds(0, _CAP)])
    pltpu.sync_copy(cnt_buf, sh.at[s, pl.ds(_CAP, _L)])
    plsc.subcore_barrier()

    d = s // 2
    h2 = s % 2
    img_loc = d // 2
    dirn = d % 2
    u_a = img_loc * 2 + dirn
    u_b = img_loc * 2 + (1 - dirn)
    row_a = u_a * 2 + h2
    r_b0 = u_b * 2
    r_b1 = u_b * 2 + 1

    for i in range(_NS):
        pltpu.sync_copy(sh.at[i, pl.ds(_CAP, _L)], counts_v.at[i])
    pltpu.sync_copy(sh.at[row_a, pl.ds(0, _CAP)], a_idx)
    pltpu.sync_copy(sh.at[r_b0, pl.ds(0, _CAP)], b_idx0)
    pltpu.sync_copy(sh.at[r_b1, pl.ds(0, _CAP)], b_idx1)

    c_ah = jnp.max(counts_v[row_a, :])
    c_a = jnp.max(counts_v[u_a * 2, :]) + jnp.max(counts_v[u_a * 2 + 1, :])
    c_b0 = jnp.max(counts_v[r_b0, :])
    c_b1 = jnp.max(counts_v[r_b1, :])
    c_b = c_b0 + c_b1

    _decode(a_idx, a_i, a_j, c_ah)
    _decode(b_idx0, b_i0, b_j0, c_b0)
    _decode(b_idx1, b_i1, b_j1, c_b1)

    res_iv[...] = plsc.bitcast(jnp.zeros((_L,), jnp.float32), jnp.int32)
    inner = ((b_i0, b_j0, c_b0), (b_i1, b_j1, c_b1))

    @pl.when((c_a > 0) & (c_b > 0))
    def _():
        res_iv[...] = plsc.bitcast(
            _pair_loop(a_i, a_j, c_ah, inner, True), jnp.int32)

    @pl.when((c_a == 0) & (c_b > 0) & (h2 == 0))
    def _():
        res_iv[...] = plsc.bitcast(
            _pair_loop(b_i0, b_j0, c_b0, inner, False), jnp.int32)

    @pl.when((c_a == 0) & (c_b > 0) & (h2 == 1))
    def _():
        res_iv[...] = plsc.bitcast(
            _pair_loop(b_i1, b_j1, c_b1, inner, False), jnp.int32)

    pltpu.sync_copy(res_iv, sh.at[s, pl.ds(_CAP + _L, _L)])
    plsc.subcore_barrier()

    @pl.when(s == 0)
    def _():
        for i in range(_NS):
            pltpu.sync_copy(sh.at[i, pl.ds(_CAP + _L, _L)], res_all.at[i])
        out_vec = jnp.zeros((_L,), jnp.float32)
        for i in range(_IPC):
            rows = [plsc.bitcast(res_all[4 * i + t, :], jnp.float32)
                    for t in range(4)]
            v = jnp.maximum(jnp.maximum(rows[0], rows[1]),
                            jnp.maximum(rows[2], rows[3]))
            out_vec = jnp.where(lanes == i, jnp.max(v), out_vec)
        out_v[...] = out_vec
        pltpu.sync_copy(out_v, out_hbm.at[c])


def kernel(prediction, target):
    mesh = plsc.VectorSubcoreMesh(
        core_axis_name="c", subcore_axis_name="s",
        num_cores=_NC, num_subcores=_NS,
    )
    out = pl.kernel(
        _hausdorff_sq_kernel,
        out_type=jax.ShapeDtypeStruct((_NC, _L), jnp.float32),
        mesh=mesh,
        compiler_params=pltpu.CompilerParams(needs_layout_passes=False),
        scratch_types=[
            pltpu.VMEM((_CHUNK_ROWS, _W), jnp.float32),
            pltpu.VMEM((_CAP,), jnp.int32),
            pltpu.VMEM((_L,), jnp.int32),
            pltpu.VMEM((_CAP,), jnp.int32),
            pltpu.VMEM((_CAP,), jnp.int32),
            pltpu.VMEM((_CAP,), jnp.int32),
            pltpu.VMEM((_CAP,), jnp.float32),
            pltpu.VMEM((_CAP,), jnp.float32),
            pltpu.VMEM((_CAP,), jnp.float32),
            pltpu.VMEM((_CAP,), jnp.float32),
            pltpu.VMEM((_CAP,), jnp.float32),
            pltpu.VMEM((_CAP,), jnp.float32),
            pltpu.VMEM((_NS, _L), jnp.int32),
            pltpu.VMEM((_L,), jnp.int32),
            pltpu.VMEM((_NS, _L), jnp.int32),
            pltpu.VMEM((_L,), jnp.float32),
            pltpu.VMEM_SHARED((_NS, _CAP + 2 * _L), jnp.int32),
        ],
    )(jnp.stack([prediction, target]))
    vals = out[:, :_IPC].reshape(_B)
    return jnp.mean(jnp.sqrt(vals))


# speedup vs baseline: 57.5721x; 1.4416x over previous
import functools

import jax
import jax.numpy as jnp
from jax import lax
from jax.experimental import pallas as pl
from jax.experimental.pallas import tpu as pltpu
from jax.experimental.pallas import tpu_sc as plsc

_TAU = 0.995
_B, _H, _W = 8, 512, 512
_NC, _NS, _L = 2, 16, 16
_IPC = _B // _NC
_CAP = 8192
_HALF_ROWS = _H // 2
_CHUNK_ROWS = 32
_NCHUNKS = _HALF_ROWS // _CHUNK_ROWS
_VPR = _W // _L
_BLK = 8


def _decode(idx_ref, i_ref, j_ref, n):
    q0 = jnp.min(idx_ref[pl.ds(0, _L)])
    p0i = (q0 >> 9).astype(jnp.float32)
    p0j = (q0 & (_W - 1)).astype(jnp.float32)
    lanes = lax.iota(jnp.int32, _L)
    nv = 2 * ((n + (2 * _L - 1)) // (2 * _L))

    def body(v, carry):
        base = v * _L
        q = idx_ref[pl.ds(base, _L)]
        valid = (base + lanes) < n
        i_ref[pl.ds(base, _L)] = jnp.where(valid, (q >> 9).astype(jnp.float32), p0i)
        j_ref[pl.ds(base, _L)] = jnp.where(valid, (q & (_W - 1)).astype(jnp.float32), p0j)
        return carry

    lax.fori_loop(0, nv, body, 0)


def _pair_loop(o_i, o_j, n_out, inner, minmode):
    nv_out2 = (n_out + (2 * _L - 1)) // (2 * _L)
    init_acc = jnp.float32(1e30) if minmode else jnp.float32(-1.0)

    def outer(ov, best):
        av_i0 = o_i[pl.ds(ov * 2 * _L, _L)]
        av_j0 = o_j[pl.ds(ov * 2 * _L, _L)]
        av_i1 = o_i[pl.ds(ov * 2 * _L + _L, _L)]
        av_j1 = o_j[pl.ds(ov * 2 * _L + _L, _L)]
        acc0 = jnp.full((_L,), init_acc, jnp.float32)
        acc1 = jnp.full((_L,), init_acc, jnp.float32)

        for b_i, b_j, n_in in inner:
            nv_in = (n_in + (_L - 1)) // _L

            def ibody(iv, accs):
                acc0, acc1 = accs
                bi_vec = b_i[pl.ds(iv * _L, _L)]
                bj_vec = b_j[pl.ds(iv * _L, _L)]
                for lane in range(_L):
                    bis = bi_vec[lane]
                    bjs = bj_vec[lane]
                    di0 = av_i0 - bis
                    dj0 = av_j0 - bjs
                    d20 = di0 * di0 + dj0 * dj0
                    di1 = av_i1 - bis
                    dj1 = av_j1 - bjs
                    d21 = di1 * di1 + dj1 * dj1
                    if minmode:
                        acc0 = jnp.minimum(acc0, d20)
                        acc1 = jnp.minimum(acc1, d21)
                    else:
                        acc0 = jnp.maximum(acc0, d20)
                        acc1 = jnp.maximum(acc1, d21)
                return (acc0, acc1)

            acc0, acc1 = lax.fori_loop(0, nv_in, ibody, (acc0, acc1))
        return jnp.maximum(best, jnp.maximum(acc0, acc1))

    return lax.fori_loop(0, nv_out2, outer, jnp.full((_L,), -1.0, jnp.float32))


def _hausdorff_sq_kernel(src_hbm, out_hbm,
                         chunk, loc_idx, cnt_buf,
                         a_idx, b_idx0, b_idx1,
                         a_i, a_j, b_i0, b_j0, b_i1, b_j1,
                         counts_v, res_iv, res_all, out_v,
                         sh):
    c = lax.axis_index("c")
    s = lax.axis_index("s")

    u = s // 2
    h = s % 2
    img = c * _IPC + (u // 2)
    side = u % 2
    row0 = h * _HALF_ROWS

    lanes = lax.iota(jnp.int32, _L)
    loc_idx[pl.ds(0, _L)] = jnp.full((_L,), 0x7FFFFFFF, jnp.int32)
    offm1 = jnp.full((_L,), -1, jnp.int32)
    for chunk_i in range(_NCHUNKS):
        r_lo = row0 + chunk_i * _CHUNK_ROWS
        pltpu.sync_copy(src_hbm.at[side, img, pl.ds(r_lo, _CHUNK_ROWS)], chunk)

        def row_body(r, offm1):
            gbase = (r_lo + r) * _W + lanes
            for blk in range(_VPR // _BLK):
                ms = []
                for k in range(_BLK):
                    vec = chunk[r, pl.ds((blk * _BLK + k) * _L, _L)]
                    ms.append(vec >= _TAU)
                m_any = ms[0]
                for m in ms[1:]:
                    m_any = m_any | m

                def emit(offm1, ms=ms, blk=blk):
                    for k in range(_BLK):
                        m = ms[k]
                        pcnt = plsc.all_reduce_population_count(m)
                        pos = plsc.cumsum(m.astype(jnp.int32))
                        tgt_lane = jnp.minimum(offm1 + pos, _CAP - 1)
                        plsc.store_scatter(
                            loc_idx, [tgt_lane],
                            gbase + ((blk * _BLK + k) * _L), mask=m)
                        offm1 = offm1 + pcnt
                    return offm1

                offm1 = lax.cond(jnp.any(m_any), emit, lambda o: o, offm1)
            return offm1

        offm1 = lax.fori_loop(0, _CHUNK_ROWS, row_body, offm1)

    cnt_buf[...] = jnp.minimum(offm1 + 1, _CAP)
    pltpu.sync_copy(loc_idx, sh.at[s, pl.ds(0, _CAP)])
    pltpu.sync_copy(cnt_buf, sh.at[s, pl.ds(_CAP, _L)])
    plsc.subcore_barrier()

    d = s // 2
    h2 = s % 2
    img_loc = d // 2
    dirn = d % 2
    u_a = img_loc * 2 + dirn
    u_b = img_loc * 2 + (1 - dirn)
    row_a = u_a * 2 + h2
    r_b0 = u_b * 2
    r_b1 = u_b * 2 + 1

    for i in range(_NS):
        pltpu.sync_copy(sh.at[i, pl.ds(_CAP, _L)], counts_v.at[i])
    pltpu.sync_copy(sh.at[row_a, pl.ds(0, _CAP)], a_idx)
    pltpu.sync_copy(sh.at[r_b0, pl.ds(0, _CAP)], b_idx0)
    pltpu.sync_copy(sh.at[r_b1, pl.ds(0, _CAP)], b_idx1)

    c_ah = jnp.max(counts_v[row_a, :])
    c_a = jnp.max(counts_v[u_a * 2, :]) + jnp.max(counts_v[u_a * 2 + 1, :])
    c_b0 = jnp.max(counts_v[r_b0, :])
    c_b1 = jnp.max(counts_v[r_b1, :])
    c_b = c_b0 + c_b1

    _decode(a_idx, a_i, a_j, c_ah)
    _decode(b_idx0, b_i0, b_j0, c_b0)
    _decode(b_idx1, b_i1, b_j1, c_b1)

    res_iv[...] = plsc.bitcast(jnp.zeros((_L,), jnp.float32), jnp.int32)
    inner = ((b_i0, b_j0, c_b0), (b_i1, b_j1, c_b1))

    @pl.when((c_a > 0) & (c_b > 0))
    def _():
        res_iv[...] = plsc.bitcast(
            _pair_loop(a_i, a_j, c_ah, inner, True), jnp.int32)

    @pl.when((c_a == 0) & (c_b > 0) & (h2 == 0))
    def _():
        res_iv[...] = plsc.bitcast(
            _pair_loop(b_i0, b_j0, c_b0, inner, False), jnp.int32)

    @pl.when((c_a == 0) & (c_b > 0) & (h2 == 1))
    def _():
        res_iv[...] = plsc.bitcast(
            _pair_loop(b_i1, b_j1, c_b1, inner, False), jnp.int32)

    pltpu.sync_copy(res_iv, sh.at[s, pl.ds(_CAP + _L, _L)])
    plsc.subcore_barrier()

    @pl.when(s == 0)
    def _():
        for i in range(_NS):
            pltpu.sync_copy(sh.at[i, pl.ds(_CAP + _L, _L)], res_all.at[i])
        out_vec = jnp.zeros((_L,), jnp.float32)
        for i in range(_IPC):
            rows = [plsc.bitcast(res_all[4 * i + t, :], jnp.float32)
                    for t in range(4)]
            v = jnp.maximum(jnp.maximum(rows[0], rows[1]),
                            jnp.maximum(rows[2], rows[3]))
            out_vec = jnp.where(lanes == i, jnp.max(v), out_vec)
        out_v[...] = out_vec
        pltpu.sync_copy(out_v, out_hbm.at[c])


def kernel(prediction, target):
    mesh = plsc.VectorSubcoreMesh(
        core_axis_name="c", subcore_axis_name="s",
        num_cores=_NC, num_subcores=_NS,
    )
    out = pl.kernel(
        _hausdorff_sq_kernel,
        out_type=jax.ShapeDtypeStruct((_NC, _L), jnp.float32),
        mesh=mesh,
        compiler_params=pltpu.CompilerParams(needs_layout_passes=False),
        scratch_types=[
            pltpu.VMEM((_CHUNK_ROWS, _W), jnp.float32),
            pltpu.VMEM((_CAP,), jnp.int32),
            pltpu.VMEM((_L,), jnp.int32),
            pltpu.VMEM((_CAP,), jnp.int32),
            pltpu.VMEM((_CAP,), jnp.int32),
            pltpu.VMEM((_CAP,), jnp.int32),
            pltpu.VMEM((_CAP,), jnp.float32),
            pltpu.VMEM((_CAP,), jnp.float32),
            pltpu.VMEM((_CAP,), jnp.float32),
            pltpu.VMEM((_CAP,), jnp.float32),
            pltpu.VMEM((_CAP,), jnp.float32),
            pltpu.VMEM((_CAP,), jnp.float32),
            pltpu.VMEM((_NS, _L), jnp.int32),
            pltpu.VMEM((_L,), jnp.int32),
            pltpu.VMEM((_NS, _L), jnp.int32),
            pltpu.VMEM((_L,), jnp.float32),
            pltpu.VMEM_SHARED((_NS, _CAP + 2 * _L), jnp.int32),
        ],
    )(jnp.stack([prediction, target]))
    vals = out[:, :_IPC].reshape(_B)
    return jnp.mean(jnp.sqrt(vals))


# speedup vs baseline: 89.4509x; 1.5537x over previous
import functools

import jax
import jax.numpy as jnp
from jax import lax
from jax.experimental import pallas as pl
from jax.experimental.pallas import tpu as pltpu
from jax.experimental.pallas import tpu_sc as plsc

_TAU = 0.995
_B, _H, _W = 8, 512, 512
_NC, _NS, _L = 2, 16, 16
_IPC = _B // _NC
_CAP = 8192
_HALF_ROWS = _H // 2
_CHUNK_ROWS = 32
_NCHUNKS = _HALF_ROWS // _CHUNK_ROWS
_VPR = _W // _L
_BLK = 8
_ROFF = _CAP + 128
_SH_ROW = _ROFF + _HALF_ROWS * _L
_R0 = 40


def _decode(idx_ref, i_ref, j_ref, n):
    q0 = jnp.min(idx_ref[pl.ds(0, _L)])
    p0i = (q0 >> 9).astype(jnp.float32)
    p0j = (q0 & (_W - 1)).astype(jnp.float32)
    lanes = lax.iota(jnp.int32, _L)
    nv = 2 * ((n + (2 * _L - 1)) // (2 * _L))

    def body(v, carry):
        base = v * _L
        q = idx_ref[pl.ds(base, _L)]
        valid = (base + lanes) < n
        i_ref[pl.ds(base, _L)] = jnp.where(valid, (q >> 9).astype(jnp.float32), p0i)
        j_ref[pl.ds(base, _L)] = jnp.where(valid, (q & (_W - 1)).astype(jnp.float32), p0j)
        return carry

    lax.fori_loop(0, nv, body, 0)


def _roff(rowoff_ref, x, n_in):
    xc = jnp.minimum(x, _HALF_ROWS - 1)
    val = jnp.max(rowoff_ref[pl.ds(xc * _L, _L)])
    return jnp.where(x >= _HALF_ROWS, n_in, val)


def _pair_loop_min(o_i, o_j, n_out, inner):
    nv_out2 = (n_out + (2 * _L - 1)) // (2 * _L)
    big = jnp.float32(1e30)

    def outer(ov, best):
        av_i0 = o_i[pl.ds(ov * 2 * _L, _L)]
        av_j0 = o_j[pl.ds(ov * 2 * _L, _L)]
        av_i1 = o_i[pl.ds(ov * 2 * _L + _L, _L)]
        av_j1 = o_j[pl.ds(ov * 2 * _L + _L, _L)]
        amin_i = jnp.minimum(jnp.min(av_i0), jnp.min(av_i1)).astype(jnp.int32)
        amax_i = jnp.maximum(jnp.max(av_i0), jnp.max(av_i1)).astype(jnp.int32)

        def w_cond(c):
            return jnp.logical_not(c[3])

        def w_body(c):
            r = c[0]
            glo = jnp.maximum(amin_i - r, 0)
            ghi = jnp.minimum(amax_i + r, _H - 1)
            acc0 = jnp.full((_L,), big, jnp.float32)
            acc1 = jnp.full((_L,), big, jnp.float32)
            for b_i, b_j, n_in, ro, s0 in inner:
                lo_l = jnp.clip(glo - s0, 0, _HALF_ROWS)
                hi1_l = jnp.clip(ghi - s0 + 1, 0, _HALF_ROWS)
                st = _roff(ro, lo_l, n_in)
                en = _roff(ro, hi1_l, n_in)

                def ibody(iv, accs):
                    acc0, acc1 = accs
                    bi_vec = b_i[pl.ds(iv * _L, _L)]
                    bj_vec = b_j[pl.ds(iv * _L, _L)]
                    for lane in range(_L):
                        bis = bi_vec[lane]
                        bjs = bj_vec[lane]
                        di0 = av_i0 - bis
                        dj0 = av_j0 - bjs
                        di1 = av_i1 - bis
                        dj1 = av_j1 - bjs
                        acc0 = jnp.minimum(acc0, di0 * di0 + dj0 * dj0)
                        acc1 = jnp.minimum(acc1, di1 * di1 + dj1 * dj1)
                    return (acc0, acc1)

                acc0, acc1 = lax.fori_loop(
                    st // _L, (en + _L - 1) // _L, ibody, (acc0, acc1))

            gl_f = glo.astype(jnp.float32)
            gh_f = ghi.astype(jnp.float32)

            def validv(acc, av_i):
                t1 = jnp.maximum(av_i - gl_f + 1.0, 0.0)
                t2 = jnp.maximum(gh_f + 1.0 - av_i, 0.0)
                c1 = (glo <= 0) | (acc <= t1 * t1)
                c2 = (ghi >= _H - 1) | (acc <= t2 * t2)
                return c1 & c2

            ok = jnp.all(validv(acc0, av_i0) & validv(acc1, av_i1))
            return (r * 4, acc0, acc1, ok)

        carry = (jnp.int32(_R0), jnp.full((_L,), big, jnp.float32),
                 jnp.full((_L,), big, jnp.float32), jnp.bool_(False))
        _, acc0, acc1, _ = lax.while_loop(w_cond, w_body, carry)
        return jnp.maximum(best, jnp.maximum(acc0, acc1))

    return lax.fori_loop(0, nv_out2, outer, jnp.full((_L,), -1.0, jnp.float32))


def _pair_loop(o_i, o_j, n_out, inner, minmode):
    nv_out2 = (n_out + (2 * _L - 1)) // (2 * _L)
    init_acc = jnp.float32(1e30) if minmode else jnp.float32(-1.0)

    def outer(ov, best):
        av_i0 = o_i[pl.ds(ov * 2 * _L, _L)]
        av_j0 = o_j[pl.ds(ov * 2 * _L, _L)]
        av_i1 = o_i[pl.ds(ov * 2 * _L + _L, _L)]
        av_j1 = o_j[pl.ds(ov * 2 * _L + _L, _L)]
        acc0 = jnp.full((_L,), init_acc, jnp.float32)
        acc1 = jnp.full((_L,), init_acc, jnp.float32)

        for b_i, b_j, n_in in inner:
            nv_in = (n_in + (_L - 1)) // _L

            def ibody(iv, accs):
                acc0, acc1 = accs
                bi_vec = b_i[pl.ds(iv * _L, _L)]
                bj_vec = b_j[pl.ds(iv * _L, _L)]
                for lane in range(_L):
                    bis = bi_vec[lane]
                    bjs = bj_vec[lane]
                    di0 = av_i0 - bis
                    dj0 = av_j0 - bjs
                    d20 = di0 * di0 + dj0 * dj0
                    di1 = av_i1 - bis
                    dj1 = av_j1 - bjs
                    d21 = di1 * di1 + dj1 * dj1
                    if minmode:
                        acc0 = jnp.minimum(acc0, d20)
                        acc1 = jnp.minimum(acc1, d21)
                    else:
                        acc0 = jnp.maximum(acc0, d20)
                        acc1 = jnp.maximum(acc1, d21)
                return (acc0, acc1)

            acc0, acc1 = lax.fori_loop(0, nv_in, ibody, (acc0, acc1))
        return jnp.maximum(best, jnp.maximum(acc0, acc1))

    return lax.fori_loop(0, nv_out2, outer, jnp.full((_L,), -1.0, jnp.float32))


def _hausdorff_sq_kernel(src_hbm, out_hbm,
                         chunk, loc_idx, cnt_buf, rowoff_loc,
                         a_idx, b_idx0, b_idx1,
                         a_i, a_j, b_i0, b_j0, b_i1, b_j1,
                         roff_b0, roff_b1,
                         counts_v, res_iv, res_all, out_v,
                         sh):
    c = lax.axis_index("c")
    s = lax.axis_index("s")

    u = s // 2
    h = s % 2
    img = c * _IPC + (u // 2)
    side = u % 2
    row0 = h * _HALF_ROWS

    lanes = lax.iota(jnp.int32, _L)
    loc_idx[pl.ds(0, _L)] = jnp.full((_L,), 0x7FFFFFFF, jnp.int32)
    offm1 = jnp.full((_L,), -1, jnp.int32)
    for chunk_i in range(_NCHUNKS):
        r_lo = row0 + chunk_i * _CHUNK_ROWS
        pltpu.sync_copy(src_hbm.at[side, img, pl.ds(r_lo, _CHUNK_ROWS)], chunk)

        def row_body(r, offm1):
            gbase = (r_lo + r) * _W + lanes
            rowoff_loc[pl.ds((chunk_i * _CHUNK_ROWS + r) * _L, _L)] = offm1 + 1
            for blk in range(_VPR // _BLK):
                ms = []
                for k in range(_BLK):
                    vec = chunk[r, pl.ds((blk * _BLK + k) * _L, _L)]
                    ms.append(vec >= _TAU)
                m_any = ms[0]
                for m in ms[1:]:
                    m_any = m_any | m

                def emit(offm1, ms=ms, blk=blk):
                    for k in range(_BLK):
                        m = ms[k]
                        pcnt = plsc.all_reduce_population_count(m)
                        pos = plsc.cumsum(m.astype(jnp.int32))
                        tgt_lane = jnp.minimum(offm1 + pos, _CAP - 1)
                        plsc.store_scatter(
                            loc_idx, [tgt_lane],
                            gbase + ((blk * _BLK + k) * _L), mask=m)
                        offm1 = offm1 + pcnt
                    return offm1

                offm1 = lax.cond(jnp.any(m_any), emit, lambda o: o, offm1)
            return offm1

        offm1 = lax.fori_loop(0, _CHUNK_ROWS, row_body, offm1)

    cnt_buf[...] = jnp.minimum(offm1 + 1, _CAP)
    pltpu.sync_copy(loc_idx, sh.at[s, pl.ds(0, _CAP)])
    pltpu.sync_copy(cnt_buf, sh.at[s, pl.ds(_CAP, _L)])
    pltpu.sync_copy(rowoff_loc, sh.at[s, pl.ds(_ROFF, _HALF_ROWS * _L)])
    plsc.subcore_barrier()

    d = s // 2
    h2 = s % 2
    img_loc = d // 2
    dirn = d % 2
    u_a = img_loc * 2 + dirn
    u_b = img_loc * 2 + (1 - dirn)
    row_a = u_a * 2 + h2
    r_b0 = u_b * 2
    r_b1 = u_b * 2 + 1

    for i in range(_NS):
        pltpu.sync_copy(sh.at[i, pl.ds(_CAP, _L)], counts_v.at[i])
    pltpu.sync_copy(sh.at[row_a, pl.ds(0, _CAP)], a_idx)
    pltpu.sync_copy(sh.at[r_b0, pl.ds(0, _CAP)], b_idx0)
    pltpu.sync_copy(sh.at[r_b1, pl.ds(0, _CAP)], b_idx1)
    pltpu.sync_copy(sh.at[r_b0, pl.ds(_ROFF, _HALF_ROWS * _L)], roff_b0)
    pltpu.sync_copy(sh.at[r_b1, pl.ds(_ROFF, _HALF_ROWS * _L)], roff_b1)

    c_ah = jnp.max(counts_v[row_a, :])
    c_a = jnp.max(counts_v[u_a * 2, :]) + jnp.max(counts_v[u_a * 2 + 1, :])
    c_b0 = jnp.max(counts_v[r_b0, :])
    c_b1 = jnp.max(counts_v[r_b1, :])
    c_b = c_b0 + c_b1

    _decode(a_idx, a_i, a_j, c_ah)
    _decode(b_idx0, b_i0, b_j0, c_b0)
    _decode(b_idx1, b_i1, b_j1, c_b1)

    res_iv[...] = plsc.bitcast(jnp.zeros((_L,), jnp.float32), jnp.int32)
    inner = ((b_i0, b_j0, c_b0), (b_i1, b_j1, c_b1))
    inner_w = ((b_i0, b_j0, c_b0, roff_b0, 0),
               (b_i1, b_j1, c_b1, roff_b1, _HALF_ROWS))

    @pl.when((c_a > 0) & (c_b > 0))
    def _():
        res_iv[...] = plsc.bitcast(
            _pair_loop_min(a_i, a_j, c_ah, inner_w), jnp.int32)

    @pl.when((c_a == 0) & (c_b > 0) & (h2 == 0))
    def _():
        res_iv[...] = plsc.bitcast(
            _pair_loop(b_i0, b_j0, c_b0, inner, False), jnp.int32)

    @pl.when((c_a == 0) & (c_b > 0) & (h2 == 1))
    def _():
        res_iv[...] = plsc.bitcast(
            _pair_loop(b_i1, b_j1, c_b1, inner, False), jnp.int32)

    pltpu.sync_copy(res_iv, sh.at[s, pl.ds(_CAP + _L, _L)])
    plsc.subcore_barrier()

    @pl.when(s == 0)
    def _():
        for i in range(_NS):
            pltpu.sync_copy(sh.at[i, pl.ds(_CAP + _L, _L)], res_all.at[i])
        out_vec = jnp.zeros((_L,), jnp.float32)
        for i in range(_IPC):
            rows = [plsc.bitcast(res_all[4 * i + t, :], jnp.float32)
                    for t in range(4)]
            v = jnp.maximum(jnp.maximum(rows[0], rows[1]),
                            jnp.maximum(rows[2], rows[3]))
            out_vec = jnp.where(lanes == i, jnp.max(v), out_vec)
        out_v[...] = out_vec
        pltpu.sync_copy(out_v, out_hbm.at[c])


def kernel(prediction, target):
    mesh = plsc.VectorSubcoreMesh(
        core_axis_name="c", subcore_axis_name="s",
        num_cores=_NC, num_subcores=_NS,
    )
    out = pl.kernel(
        _hausdorff_sq_kernel,
        out_type=jax.ShapeDtypeStruct((_NC, _L), jnp.float32),
        mesh=mesh,
        compiler_params=pltpu.CompilerParams(needs_layout_passes=False),
        scratch_types=[
            pltpu.VMEM((_CHUNK_ROWS, _W), jnp.float32),
            pltpu.VMEM((_CAP,), jnp.int32),
            pltpu.VMEM((_L,), jnp.int32),
            pltpu.VMEM((_HALF_ROWS * _L,), jnp.int32),
            pltpu.VMEM((_CAP,), jnp.int32),
            pltpu.VMEM((_CAP,), jnp.int32),
            pltpu.VMEM((_CAP,), jnp.int32),
            pltpu.VMEM((_CAP,), jnp.float32),
            pltpu.VMEM((_CAP,), jnp.float32),
            pltpu.VMEM((_CAP,), jnp.float32),
            pltpu.VMEM((_CAP,), jnp.float32),
            pltpu.VMEM((_CAP,), jnp.float32),
            pltpu.VMEM((_CAP,), jnp.float32),
            pltpu.VMEM((_HALF_ROWS * _L,), jnp.int32),
            pltpu.VMEM((_HALF_ROWS * _L,), jnp.int32),
            pltpu.VMEM((_NS, _L), jnp.int32),
            pltpu.VMEM((_L,), jnp.int32),
            pltpu.VMEM((_NS, _L), jnp.int32),
            pltpu.VMEM((_L,), jnp.float32),
            pltpu.VMEM_SHARED((_NS, _SH_ROW), jnp.int32),
        ],
    )(jnp.stack([prediction, target]))
    vals = out[:, :_IPC].reshape(_B)
    return jnp.mean(jnp.sqrt(vals))


# speedup vs baseline: 94.1155x; 1.0521x over previous
import functools

import jax
import jax.numpy as jnp
from jax import lax
from jax.experimental import pallas as pl
from jax.experimental.pallas import tpu as pltpu
from jax.experimental.pallas import tpu_sc as plsc

_TAU = 0.995
_B, _H, _W = 8, 512, 512
_NC, _NS, _L = 2, 16, 16
_IPC = _B // _NC
_CAP = 8192
_HALF_ROWS = _H // 2
_CHUNK_ROWS = 32
_NCHUNKS = _HALF_ROWS // _CHUNK_ROWS
_VPR = _W // _L
_BLK = 8
_ROFF = _CAP + 128
_SH_ROW = _ROFF + _HALF_ROWS * _L
_R0 = 24


def _decode(idx_ref, i_ref, j_ref, n):
    q0 = jnp.min(idx_ref[pl.ds(0, _L)])
    p0i = (q0 >> 9).astype(jnp.float32)
    p0j = (q0 & (_W - 1)).astype(jnp.float32)
    lanes = lax.iota(jnp.int32, _L)
    nv = 2 * ((n + (2 * _L - 1)) // (2 * _L))

    def body(v, carry):
        base = v * _L
        q = idx_ref[pl.ds(base, _L)]
        valid = (base + lanes) < n
        i_ref[pl.ds(base, _L)] = jnp.where(valid, (q >> 9).astype(jnp.float32), p0i)
        j_ref[pl.ds(base, _L)] = jnp.where(valid, (q & (_W - 1)).astype(jnp.float32), p0j)
        return carry

    lax.fori_loop(0, nv, body, 0)


def _roff(rowoff_ref, x, n_in):
    xc = jnp.minimum(x, _HALF_ROWS - 1)
    val = jnp.max(rowoff_ref[pl.ds(xc * _L, _L)])
    return jnp.where(x >= _HALF_ROWS, n_in, val)


def _pair_loop_min(o_i, o_j, n_out, inner):
    nv_out2 = (n_out + (2 * _L - 1)) // (2 * _L)
    big = jnp.float32(1e30)

    def outer(ov, best):
        av_i0 = o_i[pl.ds(ov * 2 * _L, _L)]
        av_j0 = o_j[pl.ds(ov * 2 * _L, _L)]
        av_i1 = o_i[pl.ds(ov * 2 * _L + _L, _L)]
        av_j1 = o_j[pl.ds(ov * 2 * _L + _L, _L)]
        amin_i = jnp.minimum(jnp.min(av_i0), jnp.min(av_i1)).astype(jnp.int32)
        amax_i = jnp.maximum(jnp.max(av_i0), jnp.max(av_i1)).astype(jnp.int32)

        def w_cond(c):
            return jnp.logical_not(c[3])

        def w_body(c):
            r = c[0]
            glo = jnp.maximum(amin_i - r, 0)
            ghi = jnp.minimum(amax_i + r, _H - 1)
            acc0 = jnp.full((_L,), big, jnp.float32)
            acc1 = jnp.full((_L,), big, jnp.float32)
            for b_i, b_j, n_in, ro, s0 in inner:
                lo_l = jnp.clip(glo - s0, 0, _HALF_ROWS)
                hi1_l = jnp.clip(ghi - s0 + 1, 0, _HALF_ROWS)
                st = _roff(ro, lo_l, n_in)
                en = _roff(ro, hi1_l, n_in)

                def ibody(iv, accs):
                    acc0, acc1 = accs
                    bi_vec = b_i[pl.ds(iv * _L, _L)]
                    bj_vec = b_j[pl.ds(iv * _L, _L)]
                    for lane in range(_L):
                        bis = bi_vec[lane]
                        bjs = bj_vec[lane]
                        di0 = av_i0 - bis
                        dj0 = av_j0 - bjs
                        di1 = av_i1 - bis
                        dj1 = av_j1 - bjs
                        acc0 = jnp.minimum(acc0, di0 * di0 + dj0 * dj0)
                        acc1 = jnp.minimum(acc1, di1 * di1 + dj1 * dj1)
                    return (acc0, acc1)

                acc0, acc1 = lax.fori_loop(
                    st // _L, (en + _L - 1) // _L, ibody, (acc0, acc1))

            gl_f = glo.astype(jnp.float32)
            gh_f = ghi.astype(jnp.float32)

            def validv(acc, av_i):
                t1 = jnp.maximum(av_i - gl_f + 1.0, 0.0)
                t2 = jnp.maximum(gh_f + 1.0 - av_i, 0.0)
                c1 = (glo <= 0) | (acc <= t1 * t1)
                c2 = (ghi >= _H - 1) | (acc <= t2 * t2)
                return c1 & c2

            ok = jnp.all(validv(acc0, av_i0) & validv(acc1, av_i1))
            return (r * 4, acc0, acc1, ok)

        carry = (jnp.int32(_R0), jnp.full((_L,), big, jnp.float32),
                 jnp.full((_L,), big, jnp.float32), jnp.bool_(False))
        _, acc0, acc1, _ = lax.while_loop(w_cond, w_body, carry)
        return jnp.maximum(best, jnp.maximum(acc0, acc1))

    return lax.fori_loop(0, nv_out2, outer, jnp.full((_L,), -1.0, jnp.float32))


def _pair_loop(o_i, o_j, n_out, inner, minmode):
    nv_out2 = (n_out + (2 * _L - 1)) // (2 * _L)
    init_acc = jnp.float32(1e30) if minmode else jnp.float32(-1.0)

    def outer(ov, best):
        av_i0 = o_i[pl.ds(ov * 2 * _L, _L)]
        av_j0 = o_j[pl.ds(ov * 2 * _L, _L)]
        av_i1 = o_i[pl.ds(ov * 2 * _L + _L, _L)]
        av_j1 = o_j[pl.ds(ov * 2 * _L + _L, _L)]
        acc0 = jnp.full((_L,), init_acc, jnp.float32)
        acc1 = jnp.full((_L,), init_acc, jnp.float32)

        for b_i, b_j, n_in in inner:
            nv_in = (n_in + (_L - 1)) // _L

            def ibody(iv, accs):
                acc0, acc1 = accs
                bi_vec = b_i[pl.ds(iv * _L, _L)]
                bj_vec = b_j[pl.ds(iv * _L, _L)]
                for lane in range(_L):
                    bis = bi_vec[lane]
                    bjs = bj_vec[lane]
                    di0 = av_i0 - bis
                    dj0 = av_j0 - bjs
                    d20 = di0 * di0 + dj0 * dj0
                    di1 = av_i1 - bis
                    dj1 = av_j1 - bjs
                    d21 = di1 * di1 + dj1 * dj1
                    if minmode:
                        acc0 = jnp.minimum(acc0, d20)
                        acc1 = jnp.minimum(acc1, d21)
                    else:
                        acc0 = jnp.maximum(acc0, d20)
                        acc1 = jnp.maximum(acc1, d21)
                return (acc0, acc1)

            acc0, acc1 = lax.fori_loop(0, nv_in, ibody, (acc0, acc1))
        return jnp.maximum(best, jnp.maximum(acc0, acc1))

    return lax.fori_loop(0, nv_out2, outer, jnp.full((_L,), -1.0, jnp.float32))


def _hausdorff_sq_kernel(src_hbm, out_hbm,
                         chunk, loc_idx, cnt_buf, rowoff_loc,
                         a_idx, b_idx0, b_idx1,
                         a_i, a_j, b_i0, b_j0, b_i1, b_j1,
                         roff_b0, roff_b1,
                         counts_v, res_iv, res_all, out_v,
                         sh):
    c = lax.axis_index("c")
    s = lax.axis_index("s")

    u = s // 2
    h = s % 2
    img = c * _IPC + (u // 2)
    side = u % 2
    row0 = h * _HALF_ROWS

    lanes = lax.iota(jnp.int32, _L)
    loc_idx[pl.ds(0, _L)] = jnp.full((_L,), 0x7FFFFFFF, jnp.int32)
    offm1 = jnp.full((_L,), -1, jnp.int32)
    for chunk_i in range(_NCHUNKS):
        r_lo = row0 + chunk_i * _CHUNK_ROWS
        pltpu.sync_copy(src_hbm.at[side, img, pl.ds(r_lo, _CHUNK_ROWS)], chunk)

        def row_body(r, offm1):
            gbase = (r_lo + r) * _W + lanes
            rowoff_loc[pl.ds((chunk_i * _CHUNK_ROWS + r) * _L, _L)] = offm1 + 1
            for blk in range(_VPR // _BLK):
                ms = []
                for k in range(_BLK):
                    vec = chunk[r, pl.ds((blk * _BLK + k) * _L, _L)]
                    ms.append(vec >= _TAU)
                m_any = ms[0]
                for m in ms[1:]:
                    m_any = m_any | m

                def emit(offm1, ms=ms, blk=blk):
                    for k in range(_BLK):
                        m = ms[k]
                        pcnt = plsc.all_reduce_population_count(m)
                        pos = plsc.cumsum(m.astype(jnp.int32))
                        tgt_lane = jnp.minimum(offm1 + pos, _CAP - 1)
                        plsc.store_scatter(
                            loc_idx, [tgt_lane],
                            gbase + ((blk * _BLK + k) * _L), mask=m)
                        offm1 = offm1 + pcnt
                    return offm1

                offm1 = lax.cond(jnp.any(m_any), emit, lambda o: o, offm1)
            return offm1

        offm1 = lax.fori_loop(0, _CHUNK_ROWS, row_body, offm1)

    cnt_buf[...] = jnp.minimum(offm1 + 1, _CAP)
    pltpu.sync_copy(loc_idx, sh.at[s, pl.ds(0, _CAP)])
    pltpu.sync_copy(cnt_buf, sh.at[s, pl.ds(_CAP, _L)])
    pltpu.sync_copy(rowoff_loc, sh.at[s, pl.ds(_ROFF, _HALF_ROWS * _L)])
    plsc.subcore_barrier()

    d = s // 2
    h2 = s % 2
    img_loc = d // 2
    dirn = d % 2
    u_a = img_loc * 2 + dirn
    u_b = img_loc * 2 + (1 - dirn)
    row_a = u_a * 2 + h2
    r_b0 = u_b * 2
    r_b1 = u_b * 2 + 1

    for i in range(_NS):
        pltpu.sync_copy(sh.at[i, pl.ds(_CAP, _L)], counts_v.at[i])
    pltpu.sync_copy(sh.at[row_a, pl.ds(0, _CAP)], a_idx)
    pltpu.sync_copy(sh.at[r_b0, pl.ds(0, _CAP)], b_idx0)
    pltpu.sync_copy(sh.at[r_b1, pl.ds(0, _CAP)], b_idx1)
    pltpu.sync_copy(sh.at[r_b0, pl.ds(_ROFF, _HALF_ROWS * _L)], roff_b0)
    pltpu.sync_copy(sh.at[r_b1, pl.ds(_ROFF, _HALF_ROWS * _L)], roff_b1)

    c_ah = jnp.max(counts_v[row_a, :])
    c_a = jnp.max(counts_v[u_a * 2, :]) + jnp.max(counts_v[u_a * 2 + 1, :])
    c_b0 = jnp.max(counts_v[r_b0, :])
    c_b1 = jnp.max(counts_v[r_b1, :])
    c_b = c_b0 + c_b1

    _decode(a_idx, a_i, a_j, c_ah)
    _decode(b_idx0, b_i0, b_j0, c_b0)
    _decode(b_idx1, b_i1, b_j1, c_b1)

    res_iv[...] = plsc.bitcast(jnp.zeros((_L,), jnp.float32), jnp.int32)
    inner = ((b_i0, b_j0, c_b0), (b_i1, b_j1, c_b1))
    inner_w = ((b_i0, b_j0, c_b0, roff_b0, 0),
               (b_i1, b_j1, c_b1, roff_b1, _HALF_ROWS))

    @pl.when((c_a > 0) & (c_b > 0))
    def _():
        res_iv[...] = plsc.bitcast(
            _pair_loop_min(a_i, a_j, c_ah, inner_w), jnp.int32)

    @pl.when((c_a == 0) & (c_b > 0) & (h2 == 0))
    def _():
        res_iv[...] = plsc.bitcast(
            _pair_loop(b_i0, b_j0, c_b0, inner, False), jnp.int32)

    @pl.when((c_a == 0) & (c_b > 0) & (h2 == 1))
    def _():
        res_iv[...] = plsc.bitcast(
            _pair_loop(b_i1, b_j1, c_b1, inner, False), jnp.int32)

    pltpu.sync_copy(res_iv, sh.at[s, pl.ds(_CAP + _L, _L)])
    plsc.subcore_barrier()

    @pl.when(s == 0)
    def _():
        for i in range(_NS):
            pltpu.sync_copy(sh.at[i, pl.ds(_CAP + _L, _L)], res_all.at[i])
        out_vec = jnp.zeros((_L,), jnp.float32)
        for i in range(_IPC):
            rows = [plsc.bitcast(res_all[4 * i + t, :], jnp.float32)
                    for t in range(4)]
            v = jnp.maximum(jnp.maximum(rows[0], rows[1]),
                            jnp.maximum(rows[2], rows[3]))
            out_vec = jnp.where(lanes == i, jnp.max(v), out_vec)
        out_v[...] = out_vec
        pltpu.sync_copy(out_v, out_hbm.at[c])


def kernel(prediction, target):
    mesh = plsc.VectorSubcoreMesh(
        core_axis_name="c", subcore_axis_name="s",
        num_cores=_NC, num_subcores=_NS,
    )
    out = pl.kernel(
        _hausdorff_sq_kernel,
        out_type=jax.ShapeDtypeStruct((_NC, _L), jnp.float32),
        mesh=mesh,
        compiler_params=pltpu.CompilerParams(needs_layout_passes=False),
        scratch_types=[
            pltpu.VMEM((_CHUNK_ROWS, _W), jnp.float32),
            pltpu.VMEM((_CAP,), jnp.int32),
            pltpu.VMEM((_L,), jnp.int32),
            pltpu.VMEM((_HALF_ROWS * _L,), jnp.int32),
            pltpu.VMEM((_CAP,), jnp.int32),
            pltpu.VMEM((_CAP,), jnp.int32),
            pltpu.VMEM((_CAP,), jnp.int32),
            pltpu.VMEM((_CAP,), jnp.float32),
            pltpu.VMEM((_CAP,), jnp.float32),
            pltpu.VMEM((_CAP,), jnp.float32),
            pltpu.VMEM((_CAP,), jnp.float32),
            pltpu.VMEM((_CAP,), jnp.float32),
            pltpu.VMEM((_CAP,), jnp.float32),
            pltpu.VMEM((_HALF_ROWS * _L,), jnp.int32),
            pltpu.VMEM((_HALF_ROWS * _L,), jnp.int32),
            pltpu.VMEM((_NS, _L), jnp.int32),
            pltpu.VMEM((_L,), jnp.int32),
            pltpu.VMEM((_NS, _L), jnp.int32),
            pltpu.VMEM((_L,), jnp.float32),
            pltpu.VMEM_SHARED((_NS, _SH_ROW), jnp.int32),
        ],
    )(jnp.stack([prediction, target]))
    vals = out[:, :_IPC].reshape(_B)
    return jnp.mean(jnp.sqrt(vals))
